# baseline (speedup 1.0000x reference)
import jax
import jax.numpy as jnp
from jax.experimental import pallas as pl

N_GRAPHS_ = 16
RATIO_ = 0.5


def _node_info_score(x, row, col, ew):
    num_nodes = x.shape[0]
    deg = jnp.zeros((num_nodes,), dtype=x.dtype).at[row].add(ew)
    dis = jnp.where(deg > 0, deg ** -0.5, 0.0)
    coef = -dis[row] * dis[col] * ew
    agg = jnp.zeros_like(x).at[col].add(coef[:, None] * x[row])
    return agg + x


def _topk_mask(score, batch, valid, ratio, num_graphs):
    n = score.shape[0]
    vcnt = jax.ops.segment_sum(valid.astype(score.dtype), batch, num_segments=num_graphs)
    k = jnp.ceil(ratio * vcnt).astype(jnp.int32)
    sort_score = jnp.where(valid, score, -jnp.inf)
    order = jnp.lexsort((-sort_score, batch))
    bsort = batch[order]
    total = jax.ops.segment_sum(jnp.ones((n,), dtype=jnp.int32), batch, num_segments=num_graphs)
    start = jnp.concatenate(
        [jnp.zeros((1,), dtype=jnp.int32), jnp.cumsum(total)[:-1].astype(jnp.int32)]
    )
    rank = jnp.arange(n, dtype=jnp.int32) - start[bsort]
    keep_sorted = valid[order] & (rank < k[bsort])
    return jnp.zeros((n,), dtype=bool).at[order].set(keep_sorted)


def _pool(x, edge_index, batch, valid, emask, ratio, num_graphs):
    row, col = edge_index[0], edge_index[1]
    ew = (emask & (row != col) & valid[row] & valid[col]).astype(x.dtype)
    info = _node_info_score(x, row, col, ew)
    score = jnp.sum(jnp.abs(info), axis=1)
    new_valid = _topk_mask(score, batch, valid, ratio, num_graphs)
    new_emask = emask & new_valid[row] & new_valid[col]
    return new_valid, new_emask


def _gmp_gap(x, batch, valid, num_graphs):
    ones = valid.astype(x.dtype)
    cnt = jax.ops.segment_sum(ones, batch, num_segments=num_graphs)
    sm = jax.ops.segment_sum(x * ones[:, None], batch, num_segments=num_graphs)
    mean = sm / jnp.maximum(cnt, 1.0)[:, None]
    xm = jnp.where(valid[:, None], x, -jnp.inf)
    mx = jax.ops.segment_max(xm, batch, num_segments=num_graphs)
    mx = jnp.where(cnt[:, None] > 0, mx, 0.0)
    return jnp.concatenate([mx, mean], axis=1)


def _identity_pallas(y):
    def body(y_ref, o_ref):
        o_ref[...] = y_ref[...]
    return pl.pallas_call(body, out_shape=jax.ShapeDtypeStruct(y.shape, y.dtype))(y)


def kernel(x, edge_index, batch):
    num_graphs = N_GRAPHS_
    valid = jnp.ones((x.shape[0],), dtype=bool)
    emask = jnp.ones((edge_index.shape[1],), dtype=bool)
    outs = []
    for _ in range(3):
        valid, emask = _pool(x, edge_index, batch, valid, emask, RATIO_, num_graphs)
        outs.append(_gmp_gap(x, batch, valid, num_graphs))
    out = jax.nn.relu(outs[0]) + jax.nn.relu(outs[1]) + jax.nn.relu(outs[2])
    return _identity_pallas(out)


# speedup vs baseline: 14.2791x; 14.2791x over previous
import functools

import jax
import jax.numpy as jnp
from jax import lax
from jax.experimental import pallas as pl
from jax.experimental.pallas import tpu as pltpu
from jax.experimental.pallas import tpu_sc as plsc

N = 10000
E = 320000
D = 128
G = 16
NPAD = 10112
NC = 2
NS = 16
NW = NC * NS
EPT = 10112
E_PAD = NW * EPT
KC = 128
NCHUNK = EPT // KC
STRIPE = NPAD // NS
DW = 128

_MESH = dict(core_axis_name="c", subcore_axis_name="s",
             num_cores=NC, num_subcores=NS)


def _sc_deg_body(row_hbm, col_hbm, valid_hbm, zeros8_hbm, ones8_hbm,
                 degp_hbm, ceff_hbm,
                 row_v, col_v, vr, vc, ridx, ceff_v, ones_v, deg_sh, sem):
    c = lax.axis_index("c")
    s = lax.axis_index("s")
    wid = s * NC + c
    base = wid * EPT
    pltpu.sync_copy(zeros8_hbm.at[pl.ds(s * STRIPE, STRIPE)],
                    deg_sh.at[pl.ds(s * STRIPE, STRIPE)])
    pltpu.sync_copy(row_hbm.at[pl.ds(base, EPT)], row_v)
    pltpu.sync_copy(col_hbm.at[pl.ds(base, EPT)], col_v)
    pltpu.sync_copy(ones8_hbm, ones_v)
    plsc.subcore_barrier()

    lanes = lax.iota(jnp.int32, 16)

    def chunk(j, carry):
        pltpu.async_copy(valid_hbm.at[row_v.at[pl.ds(j * KC, KC)]], vr,
                         sem).wait()
        pltpu.async_copy(valid_hbm.at[col_v.at[pl.ds(j * KC, KC)]], vc,
                         sem).wait()

        def prep(i, carry2):
            off = j * KC + i * 16
            r16 = row_v[pl.ds(off, 16)]
            c16 = col_v[pl.ds(off, 16)]
            b16 = vr[pl.ds(i * 16, 16)]
            d16 = vc[pl.ds(i * 16, 16)]
            live = (r16 != c16) & (b16 > 0) & (d16 > 0)
            dummy = N + lanes
            ridx[pl.ds(i * 16, 16)] = jnp.where(live, r16, dummy)
            ceff_v[pl.ds(off, 16)] = jnp.where(live, c16, dummy)
            return carry2

        lax.fori_loop(0, KC // 16, prep, 0)
        pltpu.sync_copy(ones_v, deg_sh.at[ridx], add=True)
        return carry

    lax.fori_loop(0, NCHUNK, chunk, 0)
    plsc.subcore_barrier()
    pltpu.sync_copy(deg_sh.at[pl.ds(s * STRIPE, STRIPE)],
                    degp_hbm.at[c, pl.ds(s * STRIPE, STRIPE)])
    pltpu.sync_copy(ceff_v, ceff_hbm.at[pl.ds(base, EPT)])


@functools.cache
def _sc_deg_kernel():
    return pl.kernel(
        _sc_deg_body,
        out_type=(jax.ShapeDtypeStruct((NC, NPAD, DW), jnp.float32),
                  jax.ShapeDtypeStruct((E_PAD,), jnp.int32)),
        mesh=plsc.VectorSubcoreMesh(**_MESH),
        scratch_types=[
            pltpu.VMEM((EPT,), jnp.int32),
            pltpu.VMEM((EPT,), jnp.int32),
            pltpu.VMEM((KC,), jnp.int32),
            pltpu.VMEM((KC,), jnp.int32),
            pltpu.VMEM((KC,), jnp.int32),
            pltpu.VMEM((EPT,), jnp.int32),
            pltpu.VMEM((KC, DW), jnp.float32),
            pltpu.VMEM_SHARED((NPAD, DW), jnp.float32),
            pltpu.SemaphoreType.DMA,
        ],
    )


def _sc_agg_body(y_hbm, row_hbm, ceff_hbm, zeros_hbm, zp_hbm,
                 row_v, ceff_v, cidx, gbuf, zsh, sem):
    c = lax.axis_index("c")
    s = lax.axis_index("s")
    wid = s * NC + c
    base = wid * EPT
    pltpu.sync_copy(zeros_hbm.at[pl.ds(s * STRIPE, STRIPE)],
                    zsh.at[pl.ds(s * STRIPE, STRIPE)])
    pltpu.sync_copy(row_hbm.at[pl.ds(base, EPT)], row_v)
    pltpu.sync_copy(ceff_hbm.at[pl.ds(base, EPT)], ceff_v)
    plsc.subcore_barrier()

    def chunk(j, carry):
        pltpu.async_copy(y_hbm.at[row_v.at[pl.ds(j * KC, KC)]], gbuf,
                         sem).wait()

        def cp(i, carry2):
            cidx[pl.ds(i * 16, 16)] = ceff_v[pl.ds(j * KC + i * 16, 16)]
            return carry2

        lax.fori_loop(0, KC // 16, cp, 0)
        pltpu.sync_copy(gbuf, zsh.at[cidx], add=True)
        return carry

    lax.fori_loop(0, NCHUNK, chunk, 0)
    plsc.subcore_barrier()
    pltpu.sync_copy(zsh.at[pl.ds(s * STRIPE, STRIPE)],
                    zp_hbm.at[c, pl.ds(s * STRIPE, STRIPE)])


@functools.cache
def _sc_agg_kernel():
    return pl.kernel(
        _sc_agg_body,
        out_type=jax.ShapeDtypeStruct((NC, NPAD, D), jnp.float32),
        mesh=plsc.VectorSubcoreMesh(**_MESH),
        scratch_types=[
            pltpu.VMEM((EPT,), jnp.int32),
            pltpu.VMEM((EPT,), jnp.int32),
            pltpu.VMEM((KC,), jnp.int32),
            pltpu.VMEM((KC, D), jnp.float32),
            pltpu.VMEM_SHARED((NPAD, D), jnp.float32),
            pltpu.SemaphoreType.DMA,
        ],
    )


def _tc_prep_body(x_ref, degr_ref, y_ref):
    deg = lax.transpose(degr_ref[...], (1, 0))
    dis = jnp.where(deg > 0.0, lax.rsqrt(deg), 0.0)
    y_ref[...] = x_ref[...] * dis


def _tc_prep(xp, deg_row):
    return pl.pallas_call(
        _tc_prep_body,
        out_shape=jax.ShapeDtypeStruct((NPAD, D), jnp.float32),
    )(xp, deg_row)


_BS = 128
_NB = NPAD // _BS


def _tc_select_body(x_ref, zp_ref, degr_ref, batch_ref, valid_ref, acc_ref,
                    nv_ref, out_ref, score_s):
    f32 = jnp.float32

    def ph1(b, carry):
        r0 = pl.multiple_of(b * _BS, _BS)
        xb = x_ref[pl.ds(r0, _BS), :]
        zb = zp_ref[0, pl.ds(r0, _BS), :] + zp_ref[1, pl.ds(r0, _BS), :]
        degb = lax.transpose(degr_ref[:, pl.ds(r0, _BS)], (1, 0))
        disb = jnp.where(degb > 0.0, lax.rsqrt(degb), 0.0)
        sb = jnp.sum(jnp.abs(xb - disb * zb), axis=1, keepdims=True)
        score_s[:, pl.ds(r0, _BS)] = lax.transpose(sb, (1, 0))
        return carry

    lax.fori_loop(0, _NB, ph1, 0)

    score = score_s[...]
    sbits = lax.bitcast_convert_type(score, jnp.int32)

    batch = batch_ref[...]
    vb = valid_ref[...] > 0
    gcol = lax.broadcasted_iota(jnp.int32, (G, 1), 0)
    Mi = (batch == gcol).astype(jnp.int32)
    Mf = Mi.astype(f32)
    vf = vb.astype(f32)
    vcnt = jnp.sum(Mf * vf, axis=1, keepdims=True)
    k = jnp.right_shift(vcnt.astype(jnp.int32) + 1, 1)
    kf = k.astype(f32)

    def rbody(t, prefix):
        bit = 30 - t
        cand = prefix | jnp.left_shift(jnp.int32(1), bit)
        candn = jnp.sum(Mi * cand, axis=0, keepdims=True)
        flag = (vb & (sbits >= candn)).astype(f32)
        cnt = jnp.sum(Mf * flag, axis=1, keepdims=True)
        return jnp.where(cnt >= kf, cand, prefix)

    thr = lax.fori_loop(0, 31, rbody, jnp.zeros((G, 1), jnp.int32))
    thrn = jnp.sum(Mi * thr, axis=0, keepdims=True)
    gt = vb & (sbits > thrn)
    cntgt = jnp.sum(Mf * gt.astype(f32), axis=1, keepdims=True)
    needed = kf - cntgt

    tie = vb & (sbits == thrn)
    tief = tie.astype(f32)
    cum = tief
    sh = 1
    while sh < NPAD:
        cum = cum + jnp.concatenate(
            [jnp.zeros((1, sh), f32), cum[:, :-sh]], axis=1)
        sh *= 2
    tgraph = jnp.sum(Mf * tief, axis=1, keepdims=True)
    tgrow = lax.transpose(tgraph, (1, 0))
    ic = lax.broadcasted_iota(jnp.int32, (G, G), 0)
    ir = lax.broadcasted_iota(jnp.int32, (G, G), 1)
    lower = jnp.where(ir < ic, 1.0, 0.0).astype(f32)
    offs = jnp.sum(lower * tgrow, axis=1, keepdims=True)
    offn = jnp.sum(Mf * offs, axis=0, keepdims=True)
    rank = cum - offn
    neededn = jnp.sum(Mf * needed, axis=0, keepdims=True)
    keep = vb & (gt | (tie & (rank <= neededn)))
    nv_ref[...] = keep.astype(jnp.int32)

    keepf = keep.astype(f32)
    Mv = Mf * keepf
    cnt = jnp.sum(Mv, axis=1, keepdims=True)
    sm = jnp.dot(Mv, x_ref[...], preferred_element_type=f32)

    def ph3(b, mx_c):
        r0 = pl.multiple_of(b * _BS, _BS)
        xb = x_ref[pl.ds(r0, _BS), :]
        kb = lax.transpose(nv_ref[:, pl.ds(r0, _BS)], (1, 0))
        bb = lax.transpose(
            batch_ref[:, pl.ds(r0, _BS)].astype(f32), (1, 0))
        mx_rows = []
        for g in range(G):
            mg = (kb > 0) & (bb == float(g))
            xm = jnp.where(mg, xb, -jnp.inf)
            mx_rows.append(jnp.max(xm, axis=0, keepdims=True))
        mx_b = jnp.concatenate(mx_rows, axis=0)
        return jnp.maximum(mx_c, mx_b)

    mx = lax.fori_loop(0, _NB, ph3, jnp.full((G, D), -jnp.inf, f32))
    mean = sm / jnp.maximum(cnt, 1.0)
    mx = jnp.where(cnt > 0.0, mx, 0.0)
    pooled = jnp.concatenate([mx, mean], axis=1)
    out_ref[...] = acc_ref[...] + jnp.maximum(pooled, 0.0)


def _tc_select(xp, zp, deg_row, batch_row, valid_row, acc):
    return pl.pallas_call(
        _tc_select_body,
        out_shape=[
            jax.ShapeDtypeStruct((1, NPAD), jnp.int32),
            jax.ShapeDtypeStruct((G, 2 * D), jnp.float32),
        ],
        scratch_shapes=[
            pltpu.VMEM((1, NPAD), jnp.float32),
        ],
    )(xp, zp, deg_row, batch_row, valid_row, acc)


def kernel(x, edge_index, batch):
    f32 = jnp.float32
    i32 = jnp.int32
    epad = jnp.zeros((E_PAD - E,), i32)
    row = jnp.concatenate([edge_index[0].astype(i32), epad])
    col = jnp.concatenate([edge_index[1].astype(i32), epad])
    xp = jnp.concatenate([x, jnp.zeros((NPAD - N, D), f32)], axis=0)
    bp = jnp.concatenate([batch.astype(i32), jnp.full((NPAD - N,), G, i32)])
    batch_row = bp[None, :]
    valid_flat = jnp.concatenate(
        [jnp.ones((N,), i32), jnp.zeros((NPAD - N,), i32)])
    zeros_nd = jnp.zeros((NPAD, D), f32)
    zeros8 = jnp.zeros((NPAD, DW), f32)
    ones8 = jnp.ones((KC, DW), f32)
    acc = jnp.zeros((G, 2 * D), f32)
    for _ in range(3):
        degp, ceff = _sc_deg_kernel()(row, col, valid_flat, zeros8, ones8)
        deg_row = (degp[0, :, 0] + degp[1, :, 0])[None, :]
        y = _tc_prep(xp, deg_row)
        zp = _sc_agg_kernel()(y, row, ceff, zeros_nd)
        nv_row, acc = _tc_select(xp, zp, deg_row, batch_row,
                                 valid_flat[None, :], acc)
        valid_flat = nv_row[0]
    return acc


# speedup vs baseline: 14.2898x; 1.0007x over previous
import functools

import jax
import jax.numpy as jnp
from jax import lax
from jax.experimental import pallas as pl
from jax.experimental.pallas import tpu as pltpu
from jax.experimental.pallas import tpu_sc as plsc

N = 10000
E = 320000
D = 128
G = 16
NPAD = 10112
NC = 2
NS = 16
NW = NC * NS
EPT = 10112
E_PAD = NW * EPT
KC = 128
NCHUNK = EPT // KC
STRIPE = NPAD // NS
DW = 128

_MESH = dict(core_axis_name="c", subcore_axis_name="s",
             num_cores=NC, num_subcores=NS)


def _sc_deg_body(row_hbm, col_hbm, valid_hbm, zeros8_hbm, ones8_hbm,
                 degp_hbm, ceff_hbm,
                 row_v, col_v, vr, vc, ridx, ceff_v, ones_v, deg_sh, sem):
    c = lax.axis_index("c")
    s = lax.axis_index("s")
    wid = s * NC + c
    base = wid * EPT
    pltpu.sync_copy(zeros8_hbm.at[pl.ds(s * STRIPE, STRIPE)],
                    deg_sh.at[pl.ds(s * STRIPE, STRIPE)])
    pltpu.sync_copy(row_hbm.at[pl.ds(base, EPT)], row_v)
    pltpu.sync_copy(col_hbm.at[pl.ds(base, EPT)], col_v)
    pltpu.sync_copy(ones8_hbm, ones_v)
    plsc.subcore_barrier()

    lanes = lax.iota(jnp.int32, 16)

    def chunk(j, carry):
        pltpu.async_copy(valid_hbm.at[row_v.at[pl.ds(j * KC, KC)]], vr,
                         sem).wait()
        pltpu.async_copy(valid_hbm.at[col_v.at[pl.ds(j * KC, KC)]], vc,
                         sem).wait()

        def prep(i, carry2):
            off = j * KC + i * 16
            r16 = row_v[pl.ds(off, 16)]
            c16 = col_v[pl.ds(off, 16)]
            b16 = vr[pl.ds(i * 16, 16)]
            d16 = vc[pl.ds(i * 16, 16)]
            live = (r16 != c16) & (b16 > 0) & (d16 > 0)
            dummy = N + lanes + 16 * ((j + i) % 7)
            ridx[pl.ds(i * 16, 16)] = jnp.where(live, r16, dummy)
            ceff_v[pl.ds(off, 16)] = jnp.where(live, c16, dummy)
            return carry2

        lax.fori_loop(0, KC // 16, prep, 0)
        pltpu.sync_copy(ones_v, deg_sh.at[ridx], add=True)
        return carry

    lax.fori_loop(0, NCHUNK, chunk, 0)
    plsc.subcore_barrier()
    pltpu.sync_copy(deg_sh.at[pl.ds(s * STRIPE, STRIPE)],
                    degp_hbm.at[c, pl.ds(s * STRIPE, STRIPE)])
    pltpu.sync_copy(ceff_v, ceff_hbm.at[pl.ds(base, EPT)])


@functools.cache
def _sc_deg_kernel():
    return pl.kernel(
        _sc_deg_body,
        out_type=(jax.ShapeDtypeStruct((NC, NPAD, DW), jnp.float32),
                  jax.ShapeDtypeStruct((E_PAD,), jnp.int32)),
        mesh=plsc.VectorSubcoreMesh(**_MESH),
        scratch_types=[
            pltpu.VMEM((EPT,), jnp.int32),
            pltpu.VMEM((EPT,), jnp.int32),
            pltpu.VMEM((KC,), jnp.int32),
            pltpu.VMEM((KC,), jnp.int32),
            pltpu.VMEM((KC,), jnp.int32),
            pltpu.VMEM((EPT,), jnp.int32),
            pltpu.VMEM((KC, DW), jnp.float32),
            pltpu.VMEM_SHARED((NPAD, DW), jnp.float32),
            pltpu.SemaphoreType.DMA,
        ],
    )


def _sc_agg_body(y_hbm, row_hbm, ceff_hbm, zeros_hbm, zp_hbm,
                 row_v, ceff_v, cidx, gbuf, zsh, sem):
    c = lax.axis_index("c")
    s = lax.axis_index("s")
    wid = s * NC + c
    base = wid * EPT
    pltpu.sync_copy(zeros_hbm.at[pl.ds(s * STRIPE, STRIPE)],
                    zsh.at[pl.ds(s * STRIPE, STRIPE)])
    pltpu.sync_copy(row_hbm.at[pl.ds(base, EPT)], row_v)
    pltpu.sync_copy(ceff_hbm.at[pl.ds(base, EPT)], ceff_v)
    plsc.subcore_barrier()

    def chunk(j, carry):
        pltpu.async_copy(y_hbm.at[row_v.at[pl.ds(j * KC, KC)]], gbuf,
                         sem).wait()

        def cp(i, carry2):
            cidx[pl.ds(i * 16, 16)] = ceff_v[pl.ds(j * KC + i * 16, 16)]
            return carry2

        lax.fori_loop(0, KC // 16, cp, 0)
        pltpu.sync_copy(gbuf, zsh.at[cidx], add=True)
        return carry

    lax.fori_loop(0, NCHUNK, chunk, 0)
    plsc.subcore_barrier()
    pltpu.sync_copy(zsh.at[pl.ds(s * STRIPE, STRIPE)],
                    zp_hbm.at[c, pl.ds(s * STRIPE, STRIPE)])


@functools.cache
def _sc_agg_kernel():
    return pl.kernel(
        _sc_agg_body,
        out_type=jax.ShapeDtypeStruct((NC, NPAD, D), jnp.float32),
        mesh=plsc.VectorSubcoreMesh(**_MESH),
        scratch_types=[
            pltpu.VMEM((EPT,), jnp.int32),
            pltpu.VMEM((EPT,), jnp.int32),
            pltpu.VMEM((KC,), jnp.int32),
            pltpu.VMEM((KC, D), jnp.float32),
            pltpu.VMEM_SHARED((NPAD, D), jnp.float32),
            pltpu.SemaphoreType.DMA,
        ],
    )


def _tc_prep_body(x_ref, degr_ref, y_ref):
    deg = lax.transpose(degr_ref[...], (1, 0))
    dis = jnp.where(deg > 0.0, lax.rsqrt(deg), 0.0)
    y_ref[...] = x_ref[...] * dis


def _tc_prep(xp, deg_row):
    return pl.pallas_call(
        _tc_prep_body,
        out_shape=jax.ShapeDtypeStruct((NPAD, D), jnp.float32),
    )(xp, deg_row)


_BS = 128
_NB = NPAD // _BS


def _tc_select_body(x_ref, zp_ref, degr_ref, batch_ref, valid_ref, acc_ref,
                    nv_ref, out_ref, score_s):
    f32 = jnp.float32

    def ph1(b, carry):
        r0 = pl.multiple_of(b * _BS, _BS)
        xb = x_ref[pl.ds(r0, _BS), :]
        zb = zp_ref[0, pl.ds(r0, _BS), :] + zp_ref[1, pl.ds(r0, _BS), :]
        degb = lax.transpose(degr_ref[:, pl.ds(r0, _BS)], (1, 0))
        disb = jnp.where(degb > 0.0, lax.rsqrt(degb), 0.0)
        sb = jnp.sum(jnp.abs(xb - disb * zb), axis=1, keepdims=True)
        score_s[:, pl.ds(r0, _BS)] = lax.transpose(sb, (1, 0))
        return carry

    lax.fori_loop(0, _NB, ph1, 0)

    score = score_s[...]
    sbits = lax.bitcast_convert_type(score, jnp.int32)

    batch = batch_ref[...]
    vb = valid_ref[...] > 0
    gcol = lax.broadcasted_iota(jnp.int32, (G, 1), 0)
    Mi = (batch == gcol).astype(jnp.int32)
    Mf = Mi.astype(f32)
    vf = vb.astype(f32)
    vcnt = jnp.sum(Mf * vf, axis=1, keepdims=True)
    k = jnp.right_shift(vcnt.astype(jnp.int32) + 1, 1)
    kf = k.astype(f32)

    def rbody(t, prefix):
        bit = 30 - t
        cand = prefix | jnp.left_shift(jnp.int32(1), bit)
        candn = jnp.sum(Mi * cand, axis=0, keepdims=True)
        flag = (vb & (sbits >= candn)).astype(f32)
        cnt = jnp.sum(Mf * flag, axis=1, keepdims=True)
        return jnp.where(cnt >= kf, cand, prefix)

    thr = lax.fori_loop(0, 31, rbody, jnp.zeros((G, 1), jnp.int32))
    thrn = jnp.sum(Mi * thr, axis=0, keepdims=True)
    gt = vb & (sbits > thrn)
    cntgt = jnp.sum(Mf * gt.astype(f32), axis=1, keepdims=True)
    needed = kf - cntgt

    tie = vb & (sbits == thrn)
    tief = tie.astype(f32)
    cum = tief
    sh = 1
    while sh < NPAD:
        cum = cum + jnp.concatenate(
            [jnp.zeros((1, sh), f32), cum[:, :-sh]], axis=1)
        sh *= 2
    tgraph = jnp.sum(Mf * tief, axis=1, keepdims=True)
    tgrow = lax.transpose(tgraph, (1, 0))
    ic = lax.broadcasted_iota(jnp.int32, (G, G), 0)
    ir = lax.broadcasted_iota(jnp.int32, (G, G), 1)
    lower = jnp.where(ir < ic, 1.0, 0.0).astype(f32)
    offs = jnp.sum(lower * tgrow, axis=1, keepdims=True)
    offn = jnp.sum(Mf * offs, axis=0, keepdims=True)
    rank = cum - offn
    neededn = jnp.sum(Mf * needed, axis=0, keepdims=True)
    keep = vb & (gt | (tie & (rank <= neededn)))
    nv_ref[...] = keep.astype(jnp.int32)

    keepf = keep.astype(f32)
    Mv = Mf * keepf
    cnt = jnp.sum(Mv, axis=1, keepdims=True)
    sm = jnp.dot(Mv, x_ref[...], preferred_element_type=f32)

    def ph3(b, mx_c):
        r0 = pl.multiple_of(b * _BS, _BS)
        xb = x_ref[pl.ds(r0, _BS), :]
        kb = lax.transpose(nv_ref[:, pl.ds(r0, _BS)], (1, 0))
        bb = lax.transpose(
            batch_ref[:, pl.ds(r0, _BS)].astype(f32), (1, 0))
        mx_rows = []
        for g in range(G):
            mg = (kb > 0) & (bb == float(g))
            xm = jnp.where(mg, xb, -jnp.inf)
            mx_rows.append(jnp.max(xm, axis=0, keepdims=True))
        mx_b = jnp.concatenate(mx_rows, axis=0)
        return jnp.maximum(mx_c, mx_b)

    mx = lax.fori_loop(0, _NB, ph3, jnp.full((G, D), -jnp.inf, f32))
    mean = sm / jnp.maximum(cnt, 1.0)
    mx = jnp.where(cnt > 0.0, mx, 0.0)
    pooled = jnp.concatenate([mx, mean], axis=1)
    out_ref[...] = acc_ref[...] + jnp.maximum(pooled, 0.0)


def _tc_select(xp, zp, deg_row, batch_row, valid_row, acc):
    return pl.pallas_call(
        _tc_select_body,
        out_shape=[
            jax.ShapeDtypeStruct((1, NPAD), jnp.int32),
            jax.ShapeDtypeStruct((G, 2 * D), jnp.float32),
        ],
        scratch_shapes=[
            pltpu.VMEM((1, NPAD), jnp.float32),
        ],
    )(xp, zp, deg_row, batch_row, valid_row, acc)


def kernel(x, edge_index, batch):
    f32 = jnp.float32
    i32 = jnp.int32
    epad = jnp.zeros((E_PAD - E,), i32)
    row = jnp.concatenate([edge_index[0].astype(i32), epad])
    col = jnp.concatenate([edge_index[1].astype(i32), epad])
    xp = jnp.concatenate([x, jnp.zeros((NPAD - N, D), f32)], axis=0)
    bp = jnp.concatenate([batch.astype(i32), jnp.full((NPAD - N,), G, i32)])
    batch_row = bp[None, :]
    valid_flat = jnp.concatenate(
        [jnp.ones((N,), i32), jnp.zeros((NPAD - N,), i32)])
    zeros_nd = jnp.zeros((NPAD, D), f32)
    zeros8 = jnp.zeros((NPAD, DW), f32)
    ones8 = jnp.ones((KC, DW), f32)
    acc = jnp.zeros((G, 2 * D), f32)
    for _ in range(3):
        degp, ceff = _sc_deg_kernel()(row, col, valid_flat, zeros8, ones8)
        deg_row = (degp[0, :, 0] + degp[1, :, 0])[None, :]
        y = _tc_prep(xp, deg_row)
        zp = _sc_agg_kernel()(y, row, ceff, zeros_nd)
        nv_row, acc = _tc_select(xp, zp, deg_row, batch_row,
                                 valid_flat[None, :], acc)
        valid_flat = nv_row[0]
    return acc
